# baseline (speedup 1.0000x reference)
import jax
import jax.numpy as jnp
from jax.experimental import pallas as pl
from jax.experimental.pallas import tpu as pltpu

_BM = 1536


def _gate_gemm_kernel(x_ref, wt_ref, o_ref):
    o_ref[...] = jnp.dot(x_ref[...], wt_ref[...],
                         preferred_element_type=jnp.float32)


def kernel(hidden_states, weight):
    m, k = hidden_states.shape
    e = weight.shape[0]
    wt = weight.T
    return pl.pallas_call(
        _gate_gemm_kernel,
        grid=(pl.cdiv(m, _BM),),
        in_specs=[
            pl.BlockSpec((_BM, k), lambda i: (i, 0)),
            pl.BlockSpec((k, e), lambda i: (0, 0)),
        ],
        out_specs=pl.BlockSpec((_BM, e), lambda i: (i, 0)),
        out_shape=jax.ShapeDtypeStruct((m, e), jnp.float32),
        compiler_params=pltpu.CompilerParams(
            dimension_semantics=("arbitrary",),
        ),
    )(hidden_states, wt)


# speedup vs baseline: 1.0057x; 1.0057x over previous
import jax
import jax.numpy as jnp
from jax.experimental import pallas as pl
from jax.experimental.pallas import tpu as pltpu

_BM = 512
_NBUF = 4


def _gate_gemm_kernel(x_hbm, wt_ref, o_ref, buf_ref, sems):
    m = x_hbm.shape[0]
    nsteps = m // _BM

    def _copy(step, slot):
        return pltpu.make_async_copy(
            x_hbm.at[pl.ds(step * _BM, _BM), :],
            buf_ref.at[slot],
            sems.at[slot],
        )

    for slot in range(_NBUF):
        _copy(slot, slot).start()

    def body(outer, _):
        for j in range(_NBUF):
            step = outer * _NBUF + j
            _copy(step, j).wait()
            o_ref[pl.ds(step * _BM, _BM), :] = jnp.dot(
                buf_ref[j], wt_ref[...], preferred_element_type=jnp.float32)
            nxt = step + _NBUF

            @pl.when(nxt < nsteps)
            def _():
                _copy(nxt, j).start()
        return _

    jax.lax.fori_loop(0, nsteps // _NBUF, body, None)


def kernel(hidden_states, weight):
    m, k = hidden_states.shape
    e = weight.shape[0]
    wt = weight.T
    return pl.pallas_call(
        _gate_gemm_kernel,
        in_specs=[
            pl.BlockSpec(memory_space=pltpu.MemorySpace.HBM),
            pl.BlockSpec(memory_space=pltpu.MemorySpace.VMEM),
        ],
        out_specs=pl.BlockSpec(memory_space=pltpu.MemorySpace.VMEM),
        out_shape=jax.ShapeDtypeStruct((m, e), jnp.float32),
        scratch_shapes=[
            pltpu.VMEM((_NBUF, _BM, k), jnp.float32),
            pltpu.SemaphoreType.DMA((_NBUF,)),
        ],
    )(hidden_states, wt)
